# baseline (speedup 1.0000x reference)
import functools

import jax
import jax.numpy as jnp
from jax import lax
from jax.experimental import pallas as pl
from jax.experimental.pallas import tpu as pltpu
from jax.experimental.pallas import tpu_sc as plsc

N = 10000
D = 128
H = 15
E = 320000

NC = 2
NS = 16
NW = NC * NS
CHUNK = 128
CPT = 80
EP = NW * CPT * CHUNK
NBUF = 4
NPAD = 10112
RPT = NPAD // NS

F = 16
BM = 1000
GRID = N // BM

_sc_mesh = plsc.VectorSubcoreMesh(core_axis_name="c", subcore_axis_name="s")
_sc_params = pltpu.CompilerParams(use_tc_tiling_on_sc=False,
                                  needs_layout_passes=False)


def _zero_fill(ref, n):
    def body(i, _):
        ref[i] = jnp.zeros((F,), jnp.float32)
        return 0
    lax.fori_loop(0, n, body, 0, unroll=False)


def _round_bf16(v):
    i = plsc.bitcast(v, jnp.int32)
    r = (i + jnp.int32(0x7FFF) + ((i >> 16) & 1)) & jnp.int32(-65536)
    return plsc.bitcast(r, jnp.float32)


def _rsqrt16(d):
    i = plsc.bitcast(d, jnp.int32)
    x = plsc.bitcast(jnp.int32(0x5F3759DF) - (i >> 1), jnp.float32)
    for _ in range(3):
        x = x * (1.5 - 0.5 * d * x * x)
    return x


def _scatter_phase(y_sh, acc_sh, sidx, didx, rows, sems):
    def _gather(j, b):
        return pltpu.make_async_copy(y_sh.at[sidx.at[j]], rows.at[b], sems[b])

    for b in range(NBUF - 1):
        _gather(b, b).start()

    def chunk(j0, _):
        for b0 in range(NBUF):
            j = j0 + b0
            nb = (b0 + NBUF - 1) % NBUF

            @pl.when(j + NBUF - 1 < CPT)
            def _():
                _gather(j + NBUF - 1, nb).start()

            _gather(j, b0).wait()
            pltpu.sync_copy(rows.at[b0], acc_sh.at[didx.at[j]], add=True)
        return 0
    lax.fori_loop(0, CPT // NBUF, lambda i, c: chunk(i * NBUF, c), 0,
                  unroll=False)


@functools.partial(
    pl.kernel,
    out_type=jax.ShapeDtypeStruct((NPAD, NC * F), jnp.float32),
    mesh=_sc_mesh,
    scratch_types=[
        pltpu.VMEM((CPT, CHUNK), jnp.int32),
        pltpu.VMEM((CHUNK, F), jnp.float32),
        pltpu.VMEM((RPT, F), jnp.float32),
        pltpu.VMEM_SHARED((NPAD, F), jnp.float32),
    ],
    compiler_params=_sc_params,
)
def _sc_count(dst_hbm, out_hbm, didx, ones, zbuf, acc_sh):
    cid = lax.axis_index("c")
    sid = lax.axis_index("s")
    wid = cid * NS + sid
    lo = sid * RPT
    pltpu.sync_copy(dst_hbm.at[pl.ds(wid * CPT, CPT)], didx)

    def ob(i, _):
        ones[i] = jnp.ones((F,), jnp.float32)
        return 0
    lax.fori_loop(0, CHUNK, ob, 0, unroll=False)
    _zero_fill(zbuf, RPT)
    pltpu.sync_copy(zbuf, acc_sh.at[pl.ds(lo, RPT)])
    plsc.subcore_barrier()

    def chunk(j, _):
        pltpu.sync_copy(ones, acc_sh.at[didx.at[j]], add=True)
        return 0
    lax.fori_loop(0, CPT, chunk, 0, unroll=False)

    plsc.subcore_barrier()

    @pl.when(cid == 0)
    def _():
        pltpu.sync_copy(acc_sh.at[pl.ds(lo, RPT)],
                        out_hbm.at[pl.ds(lo, RPT), pl.ds(0, F)])

    @pl.when(cid == 1)
    def _():
        pltpu.sync_copy(acc_sh.at[pl.ds(lo, RPT)],
                        out_hbm.at[pl.ds(lo, RPT), pl.ds(F, F)])


@functools.partial(
    pl.kernel,
    out_type=jax.ShapeDtypeStruct((NPAD, 4 * F), jnp.float32),
    mesh=_sc_mesh,
    scratch_types=[
        pltpu.VMEM((CPT, CHUNK), jnp.int32),
        pltpu.VMEM((CPT, CHUNK), jnp.int32),
        pltpu.VMEM((NBUF, CHUNK, F), jnp.float32),
        pltpu.VMEM((RPT, F), jnp.float32),
        pltpu.VMEM((RPT, F), jnp.float32),
        pltpu.VMEM((RPT, F), jnp.float32),
        pltpu.VMEM((RPT, F), jnp.float32),
        pltpu.VMEM((RPT, F), jnp.float32),
        pltpu.VMEM((RPT, F), jnp.float32),
        pltpu.VMEM_SHARED((NPAD, F), jnp.float32),
        pltpu.VMEM_SHARED((NPAD, F), jnp.float32),
        [pltpu.SemaphoreType.DMA] * NBUF,
    ],
    compiler_params=_sc_params,
)
def _sc_l1(src_hbm, dst_hbm, cnt_hbm, xw_hbm, out_hbm,
           sidx, didx, rows, zbuf, c0, c1, xwb, yb, db, acc_sh, y_sh, sems):
    cid = lax.axis_index("c")
    sid = lax.axis_index("s")
    wid = cid * NS + sid
    lo = sid * RPT
    pltpu.sync_copy(src_hbm.at[pl.ds(wid * CPT, CPT)], sidx)
    pltpu.sync_copy(dst_hbm.at[pl.ds(wid * CPT, CPT)], didx)
    pltpu.sync_copy(cnt_hbm.at[pl.ds(lo, RPT), pl.ds(0, F)], c0)
    pltpu.sync_copy(cnt_hbm.at[pl.ds(lo, RPT), pl.ds(F, F)], c1)
    pltpu.sync_copy(xw_hbm.at[pl.ds(lo, RPT)], xwb)
    _zero_fill(zbuf, RPT)

    def row(i, _):
        deg = c0[i] + c1[i] + 1.0
        dv = _rsqrt16(deg)
        db[i] = dv
        yb[i] = dv * xwb[i]
        return 0
    lax.fori_loop(0, RPT, row, 0, unroll=False)

    pltpu.sync_copy(zbuf, acc_sh.at[pl.ds(lo, RPT)])
    pltpu.sync_copy(yb, y_sh.at[pl.ds(lo, RPT)])

    @pl.when(cid == 0)
    def _():
        pltpu.sync_copy(yb, out_hbm.at[pl.ds(lo, RPT), pl.ds(2 * F, F)])
        pltpu.sync_copy(db, out_hbm.at[pl.ds(lo, RPT), pl.ds(3 * F, F)])

    plsc.subcore_barrier()
    _scatter_phase(y_sh, acc_sh, sidx, didx, rows, sems)
    plsc.subcore_barrier()

    @pl.when(cid == 0)
    def _():
        pltpu.sync_copy(acc_sh.at[pl.ds(lo, RPT)],
                        out_hbm.at[pl.ds(lo, RPT), pl.ds(0, F)])

    @pl.when(cid == 1)
    def _():
        pltpu.sync_copy(acc_sh.at[pl.ds(lo, RPT)],
                        out_hbm.at[pl.ds(lo, RPT), pl.ds(F, F)])


@functools.partial(
    pl.kernel,
    out_type=jax.ShapeDtypeStruct((NPAD, 4 * F), jnp.float32),
    mesh=_sc_mesh,
    scratch_types=[
        pltpu.VMEM((CPT, CHUNK), jnp.int32),
        pltpu.VMEM((CPT, CHUNK), jnp.int32),
        pltpu.VMEM((NBUF, CHUNK, F), jnp.float32),
        pltpu.VMEM((RPT, F), jnp.float32),
        pltpu.VMEM((RPT, F), jnp.float32),
        pltpu.VMEM((RPT, F), jnp.float32),
        pltpu.VMEM((RPT, F), jnp.float32),
        pltpu.VMEM((RPT, F), jnp.float32),
        pltpu.VMEM((RPT, F), jnp.float32),
        pltpu.VMEM((F, F), jnp.float32),
        pltpu.VMEM((1, F), jnp.float32),
        pltpu.VMEM_SHARED((NPAD, F), jnp.float32),
        pltpu.VMEM_SHARED((NPAD, F), jnp.float32),
        [pltpu.SemaphoreType.DMA] * NBUF,
    ],
    compiler_params=_sc_params,
)
def _sc_l23(src_hbm, dst_hbm, prev_hbm, w_hbm, b_hbm, out_hbm,
            sidx, didx, rows, zbuf, a0, a1, ypb, db, yb, wv, bv,
            acc_sh, y_sh, sems):
    cid = lax.axis_index("c")
    sid = lax.axis_index("s")
    wid = cid * NS + sid
    lo = sid * RPT
    pltpu.sync_copy(src_hbm.at[pl.ds(wid * CPT, CPT)], sidx)
    pltpu.sync_copy(dst_hbm.at[pl.ds(wid * CPT, CPT)], didx)
    pltpu.sync_copy(prev_hbm.at[pl.ds(lo, RPT), pl.ds(0, F)], a0)
    pltpu.sync_copy(prev_hbm.at[pl.ds(lo, RPT), pl.ds(F, F)], a1)
    pltpu.sync_copy(prev_hbm.at[pl.ds(lo, RPT), pl.ds(2 * F, F)], ypb)
    pltpu.sync_copy(prev_hbm.at[pl.ds(lo, RPT), pl.ds(3 * F, F)], db)
    pltpu.sync_copy(w_hbm, wv)
    pltpu.sync_copy(b_hbm, bv)
    _zero_fill(zbuf, RPT)

    brow = bv[0]
    wrows = [_round_bf16(wv[k]) for k in range(F)]

    def row(i, _):
        dv = db[i]
        hv = jnp.maximum(dv * (a0[i] + a1[i] + ypb[i]) + brow, 0.0)
        hv = _round_bf16(hv)
        acc_v = hv[0] * wrows[0]
        for k in range(1, F):
            acc_v = acc_v + hv[k] * wrows[k]
        yb[i] = dv * acc_v
        return 0
    lax.fori_loop(0, RPT, row, 0, unroll=False)

    pltpu.sync_copy(zbuf, acc_sh.at[pl.ds(lo, RPT)])
    pltpu.sync_copy(yb, y_sh.at[pl.ds(lo, RPT)])

    @pl.when(cid == 0)
    def _():
        pltpu.sync_copy(yb, out_hbm.at[pl.ds(lo, RPT), pl.ds(2 * F, F)])
        pltpu.sync_copy(db, out_hbm.at[pl.ds(lo, RPT), pl.ds(3 * F, F)])

    plsc.subcore_barrier()
    _scatter_phase(y_sh, acc_sh, sidx, didx, rows, sems)
    plsc.subcore_barrier()

    @pl.when(cid == 0)
    def _():
        pltpu.sync_copy(acc_sh.at[pl.ds(lo, RPT)],
                        out_hbm.at[pl.ds(lo, RPT), pl.ds(0, F)])

    @pl.when(cid == 1)
    def _():
        pltpu.sync_copy(acc_sh.at[pl.ds(lo, RPT)],
                        out_hbm.at[pl.ds(lo, RPT), pl.ds(F, F)])


def _tc_pre_body(x_ref, w_ref, o_ref):
    o_ref[...] = jnp.dot(x_ref[...], w_ref[...],
                         preferred_element_type=jnp.float32)


_tc_pre = pl.pallas_call(
    _tc_pre_body,
    grid=(GRID,),
    in_specs=[
        pl.BlockSpec((BM, D), lambda i: (i, 0)),
        pl.BlockSpec((D, F), lambda i: (0, 0)),
    ],
    out_specs=pl.BlockSpec((BM, F), lambda i: (i, 0)),
    out_shape=jax.ShapeDtypeStruct((NPAD, F), jnp.float32),
)


def _tc_post_body(p_ref, b_ref, wc_ref, bc_ref, o_ref):
    p = p_ref[...]
    acc = p[:, 0:F] + p[:, F:2 * F]
    h = jnp.maximum(p[:, 3 * F:4 * F] * (acc + p[:, 2 * F:3 * F]) + b_ref[...],
                    0.0)
    o_ref[...] = jnp.dot(h, wc_ref[...],
                         preferred_element_type=jnp.float32) + bc_ref[...]


_tc_post = pl.pallas_call(
    _tc_post_body,
    grid=(GRID,),
    in_specs=[
        pl.BlockSpec((BM, 4 * F), lambda i: (i, 0)),
        pl.BlockSpec((1, F), lambda i: (0, 0)),
        pl.BlockSpec((F, D), lambda i: (0, 0)),
        pl.BlockSpec((1, D), lambda i: (0, 0)),
    ],
    out_specs=pl.BlockSpec((BM, D), lambda i: (i, 0)),
    out_shape=jax.ShapeDtypeStruct((N, D), jnp.float32),
)


def kernel(x, edge_index, W1, b1, W2, b2, W3, b3, Wc, bc):
    pad = EP - E
    srcp = jnp.concatenate([edge_index[0], jnp.zeros((pad,), jnp.int32)])
    srcp = srcp.reshape(NW * CPT, CHUNK)
    dstp = jnp.concatenate([edge_index[1], jnp.full((pad,), N, jnp.int32)])
    dstp = dstp.reshape(NW * CPT, CHUNK)

    W1p = jnp.pad(W1, ((0, 0), (0, F - H)))
    W2p = jnp.pad(W2, ((0, F - H), (0, F - H)))
    W3p = jnp.pad(W3, ((0, F - H), (0, F - H)))
    Wcp = jnp.pad(Wc, ((0, F - H), (0, 0)))
    b1p = jnp.pad(b1, (0, F - H)).reshape(1, F)
    b2p = jnp.pad(b2, (0, F - H)).reshape(1, F)
    b3p = jnp.pad(b3, (0, F - H)).reshape(1, F)
    bcp = bc.reshape(1, D)

    cnt = _sc_count(dstp)
    xw = _tc_pre(x, W1p)
    p1 = _sc_l1(srcp, dstp, cnt, xw)
    p2 = _sc_l23(srcp, dstp, p1, W2p, b1p)
    p3 = _sc_l23(srcp, dstp, p2, W3p, b2p)
    return _tc_post(p3, b3p, Wcp, bcp)


# speedup vs baseline: 1.0659x; 1.0659x over previous
import functools

import jax
import jax.numpy as jnp
from jax import lax
from jax.experimental import pallas as pl
from jax.experimental.pallas import tpu as pltpu
from jax.experimental.pallas import tpu_sc as plsc

N = 10000
D = 128
H = 15
E = 320000

NC = 2
NS = 16
NW = NC * NS
CHUNK = 128
CPT = 80
EP = NW * CPT * CHUNK
NBUF = 4
NPAD = 10112
RPT = NPAD // NS

F = 16
BM = 1000
GRID = N // BM

_sc_mesh = plsc.VectorSubcoreMesh(core_axis_name="c", subcore_axis_name="s")
_sc_params = pltpu.CompilerParams(use_tc_tiling_on_sc=False,
                                  needs_layout_passes=False)


def _zero_fill(ref, n):
    def body(i, _):
        ref[i] = jnp.zeros((F,), jnp.float32)
        return 0
    lax.fori_loop(0, n, body, 0, unroll=False)


def _round_bf16(v):
    i = plsc.bitcast(v, jnp.int32)
    r = (i + jnp.int32(0x7FFF) + ((i >> 16) & 1)) & jnp.int32(-65536)
    return plsc.bitcast(r, jnp.float32)


def _rsqrt16(d):
    i = plsc.bitcast(d, jnp.int32)
    x = plsc.bitcast(jnp.int32(0x5F3759DF) - (i >> 1), jnp.float32)
    for _ in range(3):
        x = x * (1.5 - 0.5 * d * x * x)
    return x


def _scatter_phase(y_sh, acc_sh, sidx, didx, rows, sems):
    def _gather(j, b):
        return pltpu.make_async_copy(y_sh.at[sidx.at[j]], rows.at[b], sems[b])

    for b in range(NBUF - 1):
        _gather(b, b).start()

    def chunk(j0, _):
        for b0 in range(NBUF):
            j = j0 + b0
            nb = (b0 + NBUF - 1) % NBUF

            @pl.when(j + NBUF - 1 < CPT)
            def _():
                _gather(j + NBUF - 1, nb).start()

            _gather(j, b0).wait()
            pltpu.sync_copy(rows.at[b0], acc_sh.at[didx.at[j]], add=True)
        return 0
    lax.fori_loop(0, CPT // NBUF, lambda i, c: chunk(i * NBUF, c), 0,
                  unroll=False)


@functools.partial(
    pl.kernel,
    out_type=jax.ShapeDtypeStruct((NC, NPAD, F), jnp.float32),
    mesh=_sc_mesh,
    scratch_types=[
        pltpu.VMEM((CPT, CHUNK), jnp.int32),
        pltpu.VMEM((CHUNK, F), jnp.float32),
        pltpu.VMEM((RPT, F), jnp.float32),
        pltpu.VMEM_SHARED((NPAD, F), jnp.float32),
    ],
    compiler_params=_sc_params,
)
def _sc_count(dst_hbm, out_hbm, didx, ones, zbuf, acc_sh):
    cid = lax.axis_index("c")
    sid = lax.axis_index("s")
    wid = cid * NS + sid
    lo = sid * RPT
    pltpu.sync_copy(dst_hbm.at[pl.ds(wid * CPT, CPT)], didx)

    def ob(i, _):
        ones[i] = jnp.ones((F,), jnp.float32)
        return 0
    lax.fori_loop(0, CHUNK, ob, 0, unroll=False)
    _zero_fill(zbuf, RPT)
    pltpu.sync_copy(zbuf, acc_sh.at[pl.ds(lo, RPT)])
    plsc.subcore_barrier()

    def chunk(j, _):
        pltpu.sync_copy(ones, acc_sh.at[didx.at[j]], add=True)
        return 0
    lax.fori_loop(0, CPT, chunk, 0, unroll=False)

    plsc.subcore_barrier()
    pltpu.sync_copy(acc_sh.at[pl.ds(lo, RPT)],
                    out_hbm.at[cid].at[pl.ds(lo, RPT)])


@functools.partial(
    pl.kernel,
    out_type=jax.ShapeDtypeStruct((4, NPAD, F), jnp.float32),
    mesh=_sc_mesh,
    scratch_types=[
        pltpu.VMEM((CPT, CHUNK), jnp.int32),
        pltpu.VMEM((CPT, CHUNK), jnp.int32),
        pltpu.VMEM((NBUF, CHUNK, F), jnp.float32),
        pltpu.VMEM((RPT, F), jnp.float32),
        pltpu.VMEM((RPT, F), jnp.float32),
        pltpu.VMEM((RPT, F), jnp.float32),
        pltpu.VMEM((RPT, F), jnp.float32),
        pltpu.VMEM((RPT, F), jnp.float32),
        pltpu.VMEM((RPT, F), jnp.float32),
        pltpu.VMEM_SHARED((NPAD, F), jnp.float32),
        pltpu.VMEM_SHARED((NPAD, F), jnp.float32),
        [pltpu.SemaphoreType.DMA] * NBUF,
    ],
    compiler_params=_sc_params,
)
def _sc_l1(src_hbm, dst_hbm, cnt_hbm, xw_hbm, out_hbm,
           sidx, didx, rows, zbuf, c0, c1, xwb, yb, db, acc_sh, y_sh, sems):
    cid = lax.axis_index("c")
    sid = lax.axis_index("s")
    wid = cid * NS + sid
    lo = sid * RPT
    pltpu.sync_copy(src_hbm.at[pl.ds(wid * CPT, CPT)], sidx)
    pltpu.sync_copy(dst_hbm.at[pl.ds(wid * CPT, CPT)], didx)
    pltpu.sync_copy(cnt_hbm.at[0].at[pl.ds(lo, RPT)], c0)
    pltpu.sync_copy(cnt_hbm.at[1].at[pl.ds(lo, RPT)], c1)
    pltpu.sync_copy(xw_hbm.at[pl.ds(lo, RPT)], xwb)
    _zero_fill(zbuf, RPT)

    def row(i, _):
        deg = c0[i] + c1[i] + 1.0
        dv = _rsqrt16(deg)
        db[i] = dv
        yb[i] = dv * xwb[i]
        return 0
    lax.fori_loop(0, RPT, row, 0, unroll=False)

    pltpu.sync_copy(zbuf, acc_sh.at[pl.ds(lo, RPT)])
    pltpu.sync_copy(yb, y_sh.at[pl.ds(lo, RPT)])

    @pl.when(cid == 0)
    def _():
        pltpu.sync_copy(yb, out_hbm.at[2].at[pl.ds(lo, RPT)])
        pltpu.sync_copy(db, out_hbm.at[3].at[pl.ds(lo, RPT)])

    plsc.subcore_barrier()
    _scatter_phase(y_sh, acc_sh, sidx, didx, rows, sems)
    plsc.subcore_barrier()
    pltpu.sync_copy(acc_sh.at[pl.ds(lo, RPT)],
                    out_hbm.at[cid].at[pl.ds(lo, RPT)])


@functools.partial(
    pl.kernel,
    out_type=jax.ShapeDtypeStruct((4, NPAD, F), jnp.float32),
    mesh=_sc_mesh,
    scratch_types=[
        pltpu.VMEM((CPT, CHUNK), jnp.int32),
        pltpu.VMEM((CPT, CHUNK), jnp.int32),
        pltpu.VMEM((NBUF, CHUNK, F), jnp.float32),
        pltpu.VMEM((RPT, F), jnp.float32),
        pltpu.VMEM((RPT, F), jnp.float32),
        pltpu.VMEM((RPT, F), jnp.float32),
        pltpu.VMEM((RPT, F), jnp.float32),
        pltpu.VMEM((RPT, F), jnp.float32),
        pltpu.VMEM((RPT, F), jnp.float32),
        pltpu.VMEM((F, F), jnp.float32),
        pltpu.VMEM((1, F), jnp.float32),
        pltpu.VMEM_SHARED((NPAD, F), jnp.float32),
        pltpu.VMEM_SHARED((NPAD, F), jnp.float32),
        [pltpu.SemaphoreType.DMA] * NBUF,
    ],
    compiler_params=_sc_params,
)
def _sc_l23(src_hbm, dst_hbm, prev_hbm, w_hbm, b_hbm, out_hbm,
            sidx, didx, rows, zbuf, a0, a1, ypb, db, yb, wv, bv,
            acc_sh, y_sh, sems):
    cid = lax.axis_index("c")
    sid = lax.axis_index("s")
    wid = cid * NS + sid
    lo = sid * RPT
    pltpu.sync_copy(src_hbm.at[pl.ds(wid * CPT, CPT)], sidx)
    pltpu.sync_copy(dst_hbm.at[pl.ds(wid * CPT, CPT)], didx)
    pltpu.sync_copy(prev_hbm.at[0].at[pl.ds(lo, RPT)], a0)
    pltpu.sync_copy(prev_hbm.at[1].at[pl.ds(lo, RPT)], a1)
    pltpu.sync_copy(prev_hbm.at[2].at[pl.ds(lo, RPT)], ypb)
    pltpu.sync_copy(prev_hbm.at[3].at[pl.ds(lo, RPT)], db)
    pltpu.sync_copy(w_hbm, wv)
    pltpu.sync_copy(b_hbm, bv)
    _zero_fill(zbuf, RPT)

    brow = bv[0]
    wrows = [_round_bf16(wv[k]) for k in range(F)]

    def row(i, _):
        dv = db[i]
        hv = jnp.maximum(dv * (a0[i] + a1[i] + ypb[i]) + brow, 0.0)
        hv = _round_bf16(hv)
        acc_v = hv[0] * wrows[0]
        for k in range(1, F):
            acc_v = acc_v + hv[k] * wrows[k]
        yb[i] = dv * acc_v
        return 0
    lax.fori_loop(0, RPT, row, 0, unroll=False)

    pltpu.sync_copy(zbuf, acc_sh.at[pl.ds(lo, RPT)])
    pltpu.sync_copy(yb, y_sh.at[pl.ds(lo, RPT)])

    @pl.when(cid == 0)
    def _():
        pltpu.sync_copy(yb, out_hbm.at[2].at[pl.ds(lo, RPT)])
        pltpu.sync_copy(db, out_hbm.at[3].at[pl.ds(lo, RPT)])

    plsc.subcore_barrier()
    _scatter_phase(y_sh, acc_sh, sidx, didx, rows, sems)
    plsc.subcore_barrier()
    pltpu.sync_copy(acc_sh.at[pl.ds(lo, RPT)],
                    out_hbm.at[cid].at[pl.ds(lo, RPT)])


@functools.partial(
    pl.kernel,
    out_type=jax.ShapeDtypeStruct((NPAD, 128), jnp.float32),
    mesh=_sc_mesh,
    scratch_types=[
        pltpu.VMEM((CPT, CHUNK), jnp.int32),
        pltpu.VMEM((CPT, CHUNK), jnp.int32),
        pltpu.VMEM((NBUF, CHUNK, F), jnp.float32),
        pltpu.VMEM((RPT, F), jnp.float32),
        pltpu.VMEM((RPT, F), jnp.float32),
        pltpu.VMEM((RPT, F), jnp.float32),
        pltpu.VMEM((RPT, F), jnp.float32),
        pltpu.VMEM((RPT, F), jnp.float32),
        pltpu.VMEM((RPT, F), jnp.float32),
        pltpu.VMEM((F, F), jnp.float32),
        pltpu.VMEM((1, F), jnp.float32),
        pltpu.VMEM_SHARED((NPAD, F), jnp.float32),
        pltpu.VMEM_SHARED((NPAD, F), jnp.float32),
        [pltpu.SemaphoreType.DMA] * NBUF,
    ],
    compiler_params=_sc_params,
)
def _sc_l3(src_hbm, dst_hbm, prev_hbm, w_hbm, b_hbm, out_hbm,
           sidx, didx, rows, zbuf, a0, a1, ypb, db, yb, wv, bv,
           acc_sh, y_sh, sems):
    cid = lax.axis_index("c")
    sid = lax.axis_index("s")
    wid = cid * NS + sid
    lo = sid * RPT
    pltpu.sync_copy(src_hbm.at[pl.ds(wid * CPT, CPT)], sidx)
    pltpu.sync_copy(dst_hbm.at[pl.ds(wid * CPT, CPT)], didx)
    pltpu.sync_copy(prev_hbm.at[0].at[pl.ds(lo, RPT)], a0)
    pltpu.sync_copy(prev_hbm.at[1].at[pl.ds(lo, RPT)], a1)
    pltpu.sync_copy(prev_hbm.at[2].at[pl.ds(lo, RPT)], ypb)
    pltpu.sync_copy(prev_hbm.at[3].at[pl.ds(lo, RPT)], db)
    pltpu.sync_copy(w_hbm, wv)
    pltpu.sync_copy(b_hbm, bv)
    _zero_fill(zbuf, RPT)

    brow = bv[0]
    wrows = [_round_bf16(wv[k]) for k in range(F)]

    def row(i, _):
        dv = db[i]
        hv = jnp.maximum(dv * (a0[i] + a1[i] + ypb[i]) + brow, 0.0)
        hv = _round_bf16(hv)
        acc_v = hv[0] * wrows[0]
        for k in range(1, F):
            acc_v = acc_v + hv[k] * wrows[k]
        yb[i] = dv * acc_v
        return 0
    lax.fori_loop(0, RPT, row, 0, unroll=False)

    pltpu.sync_copy(zbuf, acc_sh.at[pl.ds(lo, RPT)])
    pltpu.sync_copy(yb, y_sh.at[pl.ds(lo, RPT)])

    @pl.when(cid == 0)
    def _():
        pltpu.sync_copy(yb, out_hbm.at[pl.ds(lo, RPT), pl.ds(2 * F, F)])
        pltpu.sync_copy(db, out_hbm.at[pl.ds(lo, RPT), pl.ds(3 * F, F)])

    plsc.subcore_barrier()
    _scatter_phase(y_sh, acc_sh, sidx, didx, rows, sems)
    plsc.subcore_barrier()

    @pl.when(cid == 0)
    def _():
        pltpu.sync_copy(acc_sh.at[pl.ds(lo, RPT)],
                        out_hbm.at[pl.ds(lo, RPT), pl.ds(0, F)])

    @pl.when(cid == 1)
    def _():
        pltpu.sync_copy(acc_sh.at[pl.ds(lo, RPT)],
                        out_hbm.at[pl.ds(lo, RPT), pl.ds(F, F)])


def _tc_pre_body(x_ref, w_ref, o_ref):
    o_ref[...] = jnp.dot(x_ref[...], w_ref[...],
                         preferred_element_type=jnp.float32)


_tc_pre = pl.pallas_call(
    _tc_pre_body,
    grid=(GRID,),
    in_specs=[
        pl.BlockSpec((BM, D), lambda i: (i, 0)),
        pl.BlockSpec((D, F), lambda i: (0, 0)),
    ],
    out_specs=pl.BlockSpec((BM, F), lambda i: (i, 0)),
    out_shape=jax.ShapeDtypeStruct((NPAD, F), jnp.float32),
)


def _tc_post_body(p_ref, b_ref, wc_ref, bc_ref, o_ref):
    p = p_ref[...]
    acc = p[:, 0:F] + p[:, F:2 * F]
    h = jnp.maximum(p[:, 3 * F:4 * F] * (acc + p[:, 2 * F:3 * F]) + b_ref[...],
                    0.0)
    o_ref[...] = jnp.dot(h, wc_ref[...],
                         preferred_element_type=jnp.float32) + bc_ref[...]


_tc_post = pl.pallas_call(
    _tc_post_body,
    grid=(GRID,),
    in_specs=[
        pl.BlockSpec((BM, 128), lambda i: (i, 0)),
        pl.BlockSpec((1, F), lambda i: (0, 0)),
        pl.BlockSpec((F, D), lambda i: (0, 0)),
        pl.BlockSpec((1, D), lambda i: (0, 0)),
    ],
    out_specs=pl.BlockSpec((BM, D), lambda i: (i, 0)),
    out_shape=jax.ShapeDtypeStruct((N, D), jnp.float32),
)


def kernel(x, edge_index, W1, b1, W2, b2, W3, b3, Wc, bc):
    pad = EP - E
    srcp = jnp.concatenate([edge_index[0], jnp.zeros((pad,), jnp.int32)])
    srcp = srcp.reshape(NW * CPT, CHUNK)
    dstp = jnp.concatenate([edge_index[1], jnp.full((pad,), N, jnp.int32)])
    dstp = dstp.reshape(NW * CPT, CHUNK)

    W1p = jnp.pad(W1, ((0, 0), (0, F - H)))
    W2p = jnp.pad(W2, ((0, F - H), (0, F - H)))
    W3p = jnp.pad(W3, ((0, F - H), (0, F - H)))
    Wcp = jnp.pad(Wc, ((0, F - H), (0, 0)))
    b1p = jnp.pad(b1, (0, F - H)).reshape(1, F)
    b2p = jnp.pad(b2, (0, F - H)).reshape(1, F)
    b3p = jnp.pad(b3, (0, F - H)).reshape(1, F)
    bcp = bc.reshape(1, D)

    cnt = _sc_count(dstp)
    xw = _tc_pre(x, W1p)
    p1 = _sc_l1(srcp, dstp, cnt, xw)
    p2 = _sc_l23(srcp, dstp, p1, W2p, b1p)
    p3 = _sc_l3(srcp, dstp, p2, W3p, b2p)
    return _tc_post(p3, b3p, Wcp, bcp)


# speedup vs baseline: 1.1093x; 1.0407x over previous
import functools

import jax
import jax.numpy as jnp
from jax import lax
from jax.experimental import pallas as pl
from jax.experimental.pallas import tpu as pltpu
from jax.experimental.pallas import tpu_sc as plsc

N = 10000
D = 128
H = 15
E = 320000

NC = 2
NS = 16
NW = NC * NS
CHUNK = 128
CPT = 80
EP = NW * CPT * CHUNK
NBUF = 4
NPAD = 10112
RPT = NPAD // NS

F = 16
BM = 1000
GRID = N // BM

_sc_mesh = plsc.VectorSubcoreMesh(core_axis_name="c", subcore_axis_name="s")
_sc_params = pltpu.CompilerParams(use_tc_tiling_on_sc=False,
                                  needs_layout_passes=False)


def _zero_fill(ref, n):
    @plsc.parallel_loop(0, n, unroll=8)
    def body(i):
        ref[i] = jnp.zeros((F,), jnp.float32)


def _round_bf16(v):
    i = plsc.bitcast(v, jnp.int32)
    r = (i + jnp.int32(0x7FFF) + ((i >> 16) & 1)) & jnp.int32(-65536)
    return plsc.bitcast(r, jnp.float32)


def _rsqrt16(d):
    i = plsc.bitcast(d, jnp.int32)
    x = plsc.bitcast(jnp.int32(0x5F3759DF) - (i >> 1), jnp.float32)
    for _ in range(3):
        x = x * (1.5 - 0.5 * d * x * x)
    return x


def _scatter_phase(y_sh, acc_sh, sidx, didx, rows, sems):
    def _gather(j, b):
        return pltpu.make_async_copy(y_sh.at[sidx.at[j]], rows.at[b], sems[b])

    for b in range(NBUF - 1):
        _gather(b, b).start()

    def chunk(j0, _):
        for b0 in range(NBUF):
            j = j0 + b0
            nb = (b0 + NBUF - 1) % NBUF

            @pl.when(j + NBUF - 1 < CPT)
            def _():
                _gather(j + NBUF - 1, nb).start()

            _gather(j, b0).wait()
            pltpu.sync_copy(rows.at[b0], acc_sh.at[didx.at[j]], add=True)
        return 0
    lax.fori_loop(0, CPT // NBUF, lambda i, c: chunk(i * NBUF, c), 0,
                  unroll=False)


@functools.partial(
    pl.kernel,
    out_type=jax.ShapeDtypeStruct((NC, NPAD, F), jnp.float32),
    mesh=_sc_mesh,
    scratch_types=[
        pltpu.VMEM((CPT, CHUNK), jnp.int32),
        pltpu.VMEM((CHUNK, F), jnp.float32),
        pltpu.VMEM((RPT, F), jnp.float32),
        pltpu.VMEM_SHARED((NPAD, F), jnp.float32),
    ],
    compiler_params=_sc_params,
)
def _sc_count(dst_hbm, out_hbm, didx, ones, zbuf, acc_sh):
    cid = lax.axis_index("c")
    sid = lax.axis_index("s")
    wid = cid * NS + sid
    lo = sid * RPT
    pltpu.sync_copy(dst_hbm.at[pl.ds(wid * CPT, CPT)], didx)

    @plsc.parallel_loop(0, CHUNK, unroll=8)
    def ob(i):
        ones[i] = jnp.ones((F,), jnp.float32)
    _zero_fill(zbuf, RPT)
    pltpu.sync_copy(zbuf, acc_sh.at[pl.ds(lo, RPT)])
    plsc.subcore_barrier()

    def chunk(j, _):
        pltpu.sync_copy(ones, acc_sh.at[didx.at[j]], add=True)
        return 0
    lax.fori_loop(0, CPT, chunk, 0, unroll=False)

    plsc.subcore_barrier()
    pltpu.sync_copy(acc_sh.at[pl.ds(lo, RPT)],
                    out_hbm.at[cid].at[pl.ds(lo, RPT)])


@functools.partial(
    pl.kernel,
    out_type=jax.ShapeDtypeStruct((4, NPAD, F), jnp.float32),
    mesh=_sc_mesh,
    scratch_types=[
        pltpu.VMEM((CPT, CHUNK), jnp.int32),
        pltpu.VMEM((CPT, CHUNK), jnp.int32),
        pltpu.VMEM((NBUF, CHUNK, F), jnp.float32),
        pltpu.VMEM((RPT, F), jnp.float32),
        pltpu.VMEM((RPT, F), jnp.float32),
        pltpu.VMEM((RPT, F), jnp.float32),
        pltpu.VMEM((RPT, F), jnp.float32),
        pltpu.VMEM((RPT, F), jnp.float32),
        pltpu.VMEM((RPT, F), jnp.float32),
        pltpu.VMEM_SHARED((NPAD, F), jnp.float32),
        pltpu.VMEM_SHARED((NPAD, F), jnp.float32),
        [pltpu.SemaphoreType.DMA] * NBUF,
    ],
    compiler_params=_sc_params,
)
def _sc_l1(src_hbm, dst_hbm, cnt_hbm, xw_hbm, out_hbm,
           sidx, didx, rows, zbuf, c0, c1, xwb, yb, db, acc_sh, y_sh, sems):
    cid = lax.axis_index("c")
    sid = lax.axis_index("s")
    wid = cid * NS + sid
    lo = sid * RPT
    pltpu.sync_copy(src_hbm.at[pl.ds(wid * CPT, CPT)], sidx)
    pltpu.sync_copy(dst_hbm.at[pl.ds(wid * CPT, CPT)], didx)
    pltpu.sync_copy(cnt_hbm.at[0].at[pl.ds(lo, RPT)], c0)
    pltpu.sync_copy(cnt_hbm.at[1].at[pl.ds(lo, RPT)], c1)
    pltpu.sync_copy(xw_hbm.at[pl.ds(lo, RPT)], xwb)
    _zero_fill(zbuf, RPT)

    @plsc.parallel_loop(0, RPT, unroll=4)
    def row(i):
        deg = c0[i] + c1[i] + 1.0
        dv = _rsqrt16(deg)
        db[i] = dv
        yb[i] = dv * xwb[i]

    pltpu.sync_copy(zbuf, acc_sh.at[pl.ds(lo, RPT)])
    pltpu.sync_copy(yb, y_sh.at[pl.ds(lo, RPT)])

    @pl.when(cid == 0)
    def _():
        pltpu.sync_copy(yb, out_hbm.at[2].at[pl.ds(lo, RPT)])
        pltpu.sync_copy(db, out_hbm.at[3].at[pl.ds(lo, RPT)])

    plsc.subcore_barrier()
    _scatter_phase(y_sh, acc_sh, sidx, didx, rows, sems)
    plsc.subcore_barrier()
    pltpu.sync_copy(acc_sh.at[pl.ds(lo, RPT)],
                    out_hbm.at[cid].at[pl.ds(lo, RPT)])


@functools.partial(
    pl.kernel,
    out_type=jax.ShapeDtypeStruct((4, NPAD, F), jnp.float32),
    mesh=_sc_mesh,
    scratch_types=[
        pltpu.VMEM((CPT, CHUNK), jnp.int32),
        pltpu.VMEM((CPT, CHUNK), jnp.int32),
        pltpu.VMEM((NBUF, CHUNK, F), jnp.float32),
        pltpu.VMEM((RPT, F), jnp.float32),
        pltpu.VMEM((RPT, F), jnp.float32),
        pltpu.VMEM((RPT, F), jnp.float32),
        pltpu.VMEM((RPT, F), jnp.float32),
        pltpu.VMEM((RPT, F), jnp.float32),
        pltpu.VMEM((RPT, F), jnp.float32),
        pltpu.VMEM((F, F), jnp.float32),
        pltpu.VMEM((1, F), jnp.float32),
        pltpu.VMEM_SHARED((NPAD, F), jnp.float32),
        pltpu.VMEM_SHARED((NPAD, F), jnp.float32),
        [pltpu.SemaphoreType.DMA] * NBUF,
    ],
    compiler_params=_sc_params,
)
def _sc_l23(src_hbm, dst_hbm, prev_hbm, w_hbm, b_hbm, out_hbm,
            sidx, didx, rows, zbuf, a0, a1, ypb, db, yb, wv, bv,
            acc_sh, y_sh, sems):
    cid = lax.axis_index("c")
    sid = lax.axis_index("s")
    wid = cid * NS + sid
    lo = sid * RPT
    pltpu.sync_copy(src_hbm.at[pl.ds(wid * CPT, CPT)], sidx)
    pltpu.sync_copy(dst_hbm.at[pl.ds(wid * CPT, CPT)], didx)
    pltpu.sync_copy(prev_hbm.at[0].at[pl.ds(lo, RPT)], a0)
    pltpu.sync_copy(prev_hbm.at[1].at[pl.ds(lo, RPT)], a1)
    pltpu.sync_copy(prev_hbm.at[2].at[pl.ds(lo, RPT)], ypb)
    pltpu.sync_copy(prev_hbm.at[3].at[pl.ds(lo, RPT)], db)
    pltpu.sync_copy(w_hbm, wv)
    pltpu.sync_copy(b_hbm, bv)
    _zero_fill(zbuf, RPT)

    brow = bv[0]
    wrows = [_round_bf16(wv[k]) for k in range(F)]

    @plsc.parallel_loop(0, RPT, unroll=2)
    def row(i):
        dv = db[i]
        hv = jnp.maximum(dv * (a0[i] + a1[i] + ypb[i]) + brow, 0.0)
        hv = _round_bf16(hv)
        acc_v = hv[0] * wrows[0]
        for k in range(1, F):
            acc_v = acc_v + hv[k] * wrows[k]
        yb[i] = dv * acc_v

    pltpu.sync_copy(zbuf, acc_sh.at[pl.ds(lo, RPT)])
    pltpu.sync_copy(yb, y_sh.at[pl.ds(lo, RPT)])

    @pl.when(cid == 0)
    def _():
        pltpu.sync_copy(yb, out_hbm.at[2].at[pl.ds(lo, RPT)])
        pltpu.sync_copy(db, out_hbm.at[3].at[pl.ds(lo, RPT)])

    plsc.subcore_barrier()
    _scatter_phase(y_sh, acc_sh, sidx, didx, rows, sems)
    plsc.subcore_barrier()
    pltpu.sync_copy(acc_sh.at[pl.ds(lo, RPT)],
                    out_hbm.at[cid].at[pl.ds(lo, RPT)])


@functools.partial(
    pl.kernel,
    out_type=jax.ShapeDtypeStruct((NPAD, 128), jnp.float32),
    mesh=_sc_mesh,
    scratch_types=[
        pltpu.VMEM((CPT, CHUNK), jnp.int32),
        pltpu.VMEM((CPT, CHUNK), jnp.int32),
        pltpu.VMEM((NBUF, CHUNK, F), jnp.float32),
        pltpu.VMEM((RPT, F), jnp.float32),
        pltpu.VMEM((RPT, F), jnp.float32),
        pltpu.VMEM((RPT, F), jnp.float32),
        pltpu.VMEM((RPT, F), jnp.float32),
        pltpu.VMEM((RPT, F), jnp.float32),
        pltpu.VMEM((RPT, F), jnp.float32),
        pltpu.VMEM((F, F), jnp.float32),
        pltpu.VMEM((1, F), jnp.float32),
        pltpu.VMEM_SHARED((NPAD, F), jnp.float32),
        pltpu.VMEM_SHARED((NPAD, F), jnp.float32),
        [pltpu.SemaphoreType.DMA] * NBUF,
    ],
    compiler_params=_sc_params,
)
def _sc_l3(src_hbm, dst_hbm, prev_hbm, w_hbm, b_hbm, out_hbm,
           sidx, didx, rows, zbuf, a0, a1, ypb, db, yb, wv, bv,
           acc_sh, y_sh, sems):
    cid = lax.axis_index("c")
    sid = lax.axis_index("s")
    wid = cid * NS + sid
    lo = sid * RPT
    pltpu.sync_copy(src_hbm.at[pl.ds(wid * CPT, CPT)], sidx)
    pltpu.sync_copy(dst_hbm.at[pl.ds(wid * CPT, CPT)], didx)
    pltpu.sync_copy(prev_hbm.at[0].at[pl.ds(lo, RPT)], a0)
    pltpu.sync_copy(prev_hbm.at[1].at[pl.ds(lo, RPT)], a1)
    pltpu.sync_copy(prev_hbm.at[2].at[pl.ds(lo, RPT)], ypb)
    pltpu.sync_copy(prev_hbm.at[3].at[pl.ds(lo, RPT)], db)
    pltpu.sync_copy(w_hbm, wv)
    pltpu.sync_copy(b_hbm, bv)
    _zero_fill(zbuf, RPT)

    brow = bv[0]
    wrows = [_round_bf16(wv[k]) for k in range(F)]

    @plsc.parallel_loop(0, RPT, unroll=2)
    def row(i):
        dv = db[i]
        hv = jnp.maximum(dv * (a0[i] + a1[i] + ypb[i]) + brow, 0.0)
        hv = _round_bf16(hv)
        acc_v = hv[0] * wrows[0]
        for k in range(1, F):
            acc_v = acc_v + hv[k] * wrows[k]
        yb[i] = dv * acc_v

    pltpu.sync_copy(zbuf, acc_sh.at[pl.ds(lo, RPT)])
    pltpu.sync_copy(yb, y_sh.at[pl.ds(lo, RPT)])

    @pl.when(cid == 0)
    def _():
        pltpu.sync_copy(yb, out_hbm.at[pl.ds(lo, RPT), pl.ds(2 * F, F)])
        pltpu.sync_copy(db, out_hbm.at[pl.ds(lo, RPT), pl.ds(3 * F, F)])

    plsc.subcore_barrier()
    _scatter_phase(y_sh, acc_sh, sidx, didx, rows, sems)
    plsc.subcore_barrier()

    @pl.when(cid == 0)
    def _():
        pltpu.sync_copy(acc_sh.at[pl.ds(lo, RPT)],
                        out_hbm.at[pl.ds(lo, RPT), pl.ds(0, F)])

    @pl.when(cid == 1)
    def _():
        pltpu.sync_copy(acc_sh.at[pl.ds(lo, RPT)],
                        out_hbm.at[pl.ds(lo, RPT), pl.ds(F, F)])


def _tc_pre_body(x_ref, w_ref, o_ref):
    o_ref[...] = jnp.dot(x_ref[...], w_ref[...],
                         preferred_element_type=jnp.float32)


_tc_pre = pl.pallas_call(
    _tc_pre_body,
    grid=(GRID,),
    in_specs=[
        pl.BlockSpec((BM, D), lambda i: (i, 0)),
        pl.BlockSpec((D, F), lambda i: (0, 0)),
    ],
    out_specs=pl.BlockSpec((BM, F), lambda i: (i, 0)),
    out_shape=jax.ShapeDtypeStruct((NPAD, F), jnp.float32),
)


def _tc_post_body(p_ref, b_ref, wc_ref, bc_ref, o_ref):
    p = p_ref[...]
    acc = p[:, 0:F] + p[:, F:2 * F]
    h = jnp.maximum(p[:, 3 * F:4 * F] * (acc + p[:, 2 * F:3 * F]) + b_ref[...],
                    0.0)
    o_ref[...] = jnp.dot(h, wc_ref[...],
                         preferred_element_type=jnp.float32) + bc_ref[...]


_tc_post = pl.pallas_call(
    _tc_post_body,
    grid=(GRID,),
    in_specs=[
        pl.BlockSpec((BM, 128), lambda i: (i, 0)),
        pl.BlockSpec((1, F), lambda i: (0, 0)),
        pl.BlockSpec((F, D), lambda i: (0, 0)),
        pl.BlockSpec((1, D), lambda i: (0, 0)),
    ],
    out_specs=pl.BlockSpec((BM, D), lambda i: (i, 0)),
    out_shape=jax.ShapeDtypeStruct((N, D), jnp.float32),
)


def kernel(x, edge_index, W1, b1, W2, b2, W3, b3, Wc, bc):
    pad = EP - E
    srcp = jnp.concatenate([edge_index[0], jnp.zeros((pad,), jnp.int32)])
    srcp = srcp.reshape(NW * CPT, CHUNK)
    dstp = jnp.concatenate([edge_index[1], jnp.full((pad,), N, jnp.int32)])
    dstp = dstp.reshape(NW * CPT, CHUNK)

    W1p = jnp.pad(W1, ((0, 0), (0, F - H)))
    W2p = jnp.pad(W2, ((0, F - H), (0, F - H)))
    W3p = jnp.pad(W3, ((0, F - H), (0, F - H)))
    Wcp = jnp.pad(Wc, ((0, F - H), (0, 0)))
    b1p = jnp.pad(b1, (0, F - H)).reshape(1, F)
    b2p = jnp.pad(b2, (0, F - H)).reshape(1, F)
    b3p = jnp.pad(b3, (0, F - H)).reshape(1, F)
    bcp = bc.reshape(1, D)

    cnt = _sc_count(dstp)
    xw = _tc_pre(x, W1p)
    p1 = _sc_l1(srcp, dstp, cnt, xw)
    p2 = _sc_l23(srcp, dstp, p1, W2p, b1p)
    p3 = _sc_l3(srcp, dstp, p2, W3p, b2p)
    return _tc_post(p3, b3p, Wcp, bcp)
